# baseline (speedup 1.0000x reference)
import functools

import jax
import jax.numpy as jnp
from jax import lax
from jax.experimental import pallas as pl
from jax.experimental.pallas import tpu as pltpu
from jax.experimental.pallas import tpu_sc as plsc

NUM_K = 8192
DIM = 256
NUM_N = 16384
TN = 256
STEPS = NUM_N // TN

SC_CORES = 2
SC_SUBCORES = 16
SC_WORKERS = SC_CORES * SC_SUBCORES
ROWS_PER_W = NUM_N // SC_WORKERS
GCHUNK = 128
NCHUNK = ROWS_PER_W // GCHUNK


def _esq_body(et2_ref, esq_ref):
    et2 = et2_ref[...]
    esq_ref[...] = 0.25 * jnp.sum(et2 * et2, axis=0, keepdims=True)


def _esq_call(et2):
    return pl.pallas_call(
        _esq_body,
        out_shape=jax.ShapeDtypeStruct((1, NUM_K), jnp.float32),
    )(et2)


def _dist_body(z_ref, et2_ref, esq_in_ref, idx_ref, loss_ref, perp_ref,
               counts_ref, acc_ref):
    i = pl.program_id(0)
    et2 = et2_ref[...]

    @pl.when(i == 0)
    def _init():
        counts_ref[...] = jnp.zeros((1, NUM_K), jnp.float32)
        acc_ref[0] = 0.0

    z = z_ref[...]
    z_sq = jnp.sum(z * z, axis=1, keepdims=True)
    s2 = lax.dot_general(z, et2, (((1,), (0,)), ((), ())),
                         preferred_element_type=jnp.float32)
    d2 = (z_sq + esq_in_ref[...]) - s2
    dist = jnp.sqrt(jnp.clip(d2, 0.0, None))
    minv = jnp.min(dist, axis=1, keepdims=True)
    am = jnp.argmin(dist, axis=1)
    idx_ref[...] = am[None, None, :]

    acc_ref[0] += jnp.sum(minv * minv)
    kiota = lax.broadcasted_iota(jnp.int32, (TN, NUM_K), 1)
    idxs2 = am.reshape(TN, 1)
    counts_ref[...] += jnp.sum((idxs2 == kiota).astype(jnp.float32),
                               axis=0, keepdims=True)

    @pl.when(i == STEPS - 1)
    def _fini():
        ones11 = jnp.ones((1, 1), jnp.float32)
        loss_ref[...] = (0.1 * (acc_ref[0] / (NUM_N * DIM))) * ones11
        p = counts_ref[...] / float(NUM_N)
        ent = jnp.sum(p * jnp.log(p + 1e-10), axis=1, keepdims=True)
        perp_ref[...] = jnp.exp(-ent)


def _dist_call(z_e, et2, esq):
    return pl.pallas_call(
        _dist_body,
        grid=(STEPS,),
        in_specs=[
            pl.BlockSpec((TN, DIM), lambda i: (i, 0)),
            pl.BlockSpec((DIM, NUM_K), lambda i: (0, 0)),
            pl.BlockSpec((1, NUM_K), lambda i: (0, 0)),
        ],
        out_specs=[
            pl.BlockSpec((1, 1, TN), lambda i: (i, 0, 0)),
            pl.BlockSpec((1, 1), lambda i: (0, 0)),
            pl.BlockSpec((1, 1), lambda i: (0, 0)),
        ],
        out_shape=[
            jax.ShapeDtypeStruct((STEPS, 1, TN), jnp.int32),
            jax.ShapeDtypeStruct((1, 1), jnp.float32),
            jax.ShapeDtypeStruct((1, 1), jnp.float32),
        ],
        scratch_shapes=[
            pltpu.VMEM((1, NUM_K), jnp.float32),
            pltpu.SMEM((1,), jnp.float32),
        ],
        compiler_params=pltpu.CompilerParams(
            dimension_semantics=("arbitrary",)),
    )(z_e, et2, esq)


def _gather_body(e_hbm, idx_hbm, out_hbm, idx_c, rows_v, sem):
    c = lax.axis_index("c")
    s = lax.axis_index("s")
    wid = s * SC_CORES + c
    base = wid * ROWS_PER_W
    for j in range(NCHUNK):
        off = base + j * GCHUNK
        pltpu.sync_copy(idx_hbm.at[pl.ds(off, GCHUNK)], idx_c)
        pltpu.async_copy(e_hbm.at[idx_c], rows_v, sem).wait()
        pltpu.sync_copy(rows_v, out_hbm.at[pl.ds(off, GCHUNK)])


def _gather_call(embeddings, indices):
    mesh = plsc.VectorSubcoreMesh(core_axis_name="c", subcore_axis_name="s")
    k = functools.partial(
        pl.kernel,
        mesh=mesh,
        out_type=jax.ShapeDtypeStruct((NUM_N, DIM), jnp.float32),
        scratch_types=[
            pltpu.VMEM((GCHUNK,), jnp.int32),
            pltpu.VMEM((GCHUNK, DIM), jnp.float32),
            pltpu.SemaphoreType.DMA,
        ],
    )(_gather_body)
    return k(embeddings, indices)


def kernel(z_e, embeddings):
    et2 = (embeddings + embeddings).T
    esq = _esq_call(et2)
    idx3, loss, perp = _dist_call(z_e, et2, esq)
    indices = idx3.reshape(NUM_N)
    z_q = _gather_call(embeddings, indices)
    return z_q, indices, loss[0, 0], perp[0, 0]


# speedup vs baseline: 1.1015x; 1.1015x over previous
import functools

import jax
import jax.numpy as jnp
from jax import lax
from jax.experimental import pallas as pl
from jax.experimental.pallas import tpu as pltpu
from jax.experimental.pallas import tpu_sc as plsc

NUM_K = 8192
DIM = 256
NUM_N = 16384
TN = 512
STEPS = NUM_N // TN
KT = 4
KW = NUM_K // KT

SC_CORES = 2
SC_SUBCORES = 16
SC_WORKERS = SC_CORES * SC_SUBCORES
ROWS_PER_W = NUM_N // SC_WORKERS
GCHUNK = 128
NCHUNK = ROWS_PER_W // GCHUNK


def _esq_body(et2_ref, esq_ref):
    et2 = et2_ref[...]
    esq_ref[...] = 0.25 * jnp.sum(et2 * et2, axis=0, keepdims=True)


def _esq_call(et2):
    return pl.pallas_call(
        _esq_body,
        out_shape=jax.ShapeDtypeStruct((1, NUM_K), jnp.float32),
    )(et2)


def _dist_body(z_ref, et2_ref, esq_ref, idx_ref, loss_ref, perp_ref,
               counts_ref, acc_ref):
    i = pl.program_id(0)

    @pl.when(i == 0)
    def _init():
        counts_ref[...] = jnp.zeros((1, NUM_K), jnp.float32)
        acc_ref[0] = 0.0

    z = z_ref[...]
    z_sq = jnp.sum(z * z, axis=1, keepdims=True)

    dists = []
    lvs = []
    for kt in range(KT):
        et2_t = et2_ref[:, kt * KW:(kt + 1) * KW]
        s2 = lax.dot_general(z, et2_t, (((1,), (0,)), ((), ())),
                             preferred_element_type=jnp.float32)
        d2 = (z_sq + esq_ref[:, kt * KW:(kt + 1) * KW]) - s2
        dist = jnp.sqrt(jnp.clip(d2, 0.0, None))
        dists.append(dist)
        lvs.append(jnp.min(dist, axis=1, keepdims=True))

    minv = lvs[0]
    for kt in range(1, KT):
        minv = jnp.minimum(minv, lvs[kt])

    kio = lax.broadcasted_iota(jnp.int32, (TN, KW), 1)
    idxs = jnp.full((TN, 1), NUM_K, jnp.int32)
    for kt in range(KT):
        ci = jnp.min(jnp.where(dists[kt] == minv, kio, KW),
                     axis=1, keepdims=True)
        ci = jnp.where(ci == KW, NUM_K, ci + kt * KW)
        idxs = jnp.minimum(idxs, ci)
    idx_ref[...] = idxs

    acc_ref[0] += jnp.sum(minv * minv)
    for kt in range(KT):
        loc = idxs - kt * KW
        counts_ref[:, kt * KW:(kt + 1) * KW] += jnp.sum(
            (loc == kio).astype(jnp.float32), axis=0, keepdims=True)

    @pl.when(i == STEPS - 1)
    def _fini():
        ones11 = jnp.ones((1, 1), jnp.float32)
        loss_ref[...] = (0.1 * (acc_ref[0] / (NUM_N * DIM))) * ones11
        p = counts_ref[...] / float(NUM_N)
        ent = jnp.sum(p * jnp.log(p + 1e-10), axis=1, keepdims=True)
        perp_ref[...] = jnp.exp(-ent)


def _dist_call(z_e, et2, esq):
    return pl.pallas_call(
        _dist_body,
        grid=(STEPS,),
        in_specs=[
            pl.BlockSpec((TN, DIM), lambda i: (i, 0)),
            pl.BlockSpec((DIM, NUM_K), lambda i: (0, 0)),
            pl.BlockSpec((1, NUM_K), lambda i: (0, 0)),
        ],
        out_specs=[
            pl.BlockSpec((TN, 1), lambda i: (i, 0)),
            pl.BlockSpec((1, 1), lambda i: (0, 0)),
            pl.BlockSpec((1, 1), lambda i: (0, 0)),
        ],
        out_shape=[
            jax.ShapeDtypeStruct((NUM_N, 1), jnp.int32),
            jax.ShapeDtypeStruct((1, 1), jnp.float32),
            jax.ShapeDtypeStruct((1, 1), jnp.float32),
        ],
        scratch_shapes=[
            pltpu.VMEM((1, NUM_K), jnp.float32),
            pltpu.SMEM((1,), jnp.float32),
        ],
        compiler_params=pltpu.CompilerParams(
            dimension_semantics=("arbitrary",)),
    )(z_e, et2, esq)


def _gather_body(e_hbm, idx_hbm, out_hbm, idx_c, rows_v, sem):
    c = lax.axis_index("c")
    s = lax.axis_index("s")
    wid = s * SC_CORES + c
    base = wid * ROWS_PER_W
    for j in range(NCHUNK):
        off = base + j * GCHUNK
        pltpu.sync_copy(idx_hbm.at[pl.ds(off, GCHUNK)], idx_c)
        pltpu.async_copy(e_hbm.at[idx_c], rows_v, sem).wait()
        pltpu.sync_copy(rows_v, out_hbm.at[pl.ds(off, GCHUNK)])


def _gather_call(embeddings, indices):
    mesh = plsc.VectorSubcoreMesh(core_axis_name="c", subcore_axis_name="s")
    k = functools.partial(
        pl.kernel,
        mesh=mesh,
        out_type=jax.ShapeDtypeStruct((NUM_N, DIM), jnp.float32),
        scratch_types=[
            pltpu.VMEM((GCHUNK,), jnp.int32),
            pltpu.VMEM((GCHUNK, DIM), jnp.float32),
            pltpu.SemaphoreType.DMA,
        ],
    )(_gather_body)
    return k(embeddings, indices)


def kernel(z_e, embeddings):
    et2 = (embeddings + embeddings).T
    esq = _esq_call(et2)
    idx2, loss, perp = _dist_call(z_e, et2, esq)
    indices = idx2.reshape(NUM_N)
    z_q = _gather_call(embeddings, indices)
    return z_q, indices, loss[0, 0], perp[0, 0]
